# baseline (device time: 59545 ns/iter reference)
import jax
import jax.numpy as jnp
from jax import lax
from jax.experimental import pallas as pl
from jax.experimental.pallas import tpu as pltpu

N_Y = 4


def _ring_allgather_kv(Kf, Vf):
    b, s, f = Kf.shape

    def body(k_ref, v_ref, ko_ref, vo_ref,
             k_comm, v_comm, k_send, k_recv, v_send, v_recv):
        my_x = lax.axis_index("x")
        my_y = lax.axis_index("y")
        my_z = lax.axis_index("z")
        left = (my_x, (my_y - 1) % N_Y, my_z)
        right = (my_x, (my_y + 1) % N_Y, my_z)

        barrier_sem = pltpu.get_barrier_semaphore()
        for nbr in (left, right):
            pl.semaphore_signal(
                barrier_sem, inc=1,
                device_id=nbr, device_id_type=pl.DeviceIdType.MESH,
            )
        pl.semaphore_wait(barrier_sem, 2)

        ko_ref[:, pl.ds(my_y * s, s), :] = k_ref[...]
        vo_ref[:, pl.ds(my_y * s, s), :] = v_ref[...]
        k_comm[0] = k_ref[...]
        v_comm[0] = v_ref[...]

        for h in range(N_Y - 1):
            send_slot = h % 2
            recv_slot = (h + 1) % 2
            k_rdma = pltpu.make_async_remote_copy(
                src_ref=k_comm.at[send_slot],
                dst_ref=k_comm.at[recv_slot],
                send_sem=k_send.at[send_slot],
                recv_sem=k_recv.at[recv_slot],
                device_id=right,
                device_id_type=pl.DeviceIdType.MESH,
            )
            v_rdma = pltpu.make_async_remote_copy(
                src_ref=v_comm.at[send_slot],
                dst_ref=v_comm.at[recv_slot],
                send_sem=v_send.at[send_slot],
                recv_sem=v_recv.at[recv_slot],
                device_id=right,
                device_id_type=pl.DeviceIdType.MESH,
            )
            k_rdma.start()
            v_rdma.start()
            k_rdma.wait()
            v_rdma.wait()

            origin = (my_y - h - 1) % N_Y
            ko_ref[:, pl.ds(origin * s, s), :] = k_comm[recv_slot]
            vo_ref[:, pl.ds(origin * s, s), :] = v_comm[recv_slot]

    out_sd = jax.ShapeDtypeStruct((b, N_Y * s, f), Kf.dtype)
    return pl.pallas_call(
        body,
        out_shape=[out_sd, out_sd],
        in_specs=[
            pl.BlockSpec(memory_space=pltpu.VMEM),
            pl.BlockSpec(memory_space=pltpu.VMEM),
        ],
        out_specs=[
            pl.BlockSpec(memory_space=pltpu.VMEM),
            pl.BlockSpec(memory_space=pltpu.VMEM),
        ],
        scratch_shapes=[
            pltpu.VMEM((2, b, s, f), Kf.dtype),
            pltpu.VMEM((2, b, s, f), Vf.dtype),
            pltpu.SemaphoreType.DMA((2,)),
            pltpu.SemaphoreType.DMA((2,)),
            pltpu.SemaphoreType.DMA((2,)),
            pltpu.SemaphoreType.DMA((2,)),
        ],
        compiler_params=pltpu.CompilerParams(collective_id=0),
    )(Kf, Vf)


def kernel(Q, K, V):
    b, s, h, d = K.shape
    f = h * d

    Kf = K.astype(jnp.bfloat16).reshape(b, s, f)
    Vf = V.astype(jnp.bfloat16).reshape(b, s, f)
    K_full, V_full = _ring_allgather_kv(Kf, Vf)
    K_full = K_full.reshape(b, N_Y * s, h, d)
    V_full = V_full.reshape(b, N_Y * s, h, d)

    scale = d ** -0.5
    S = jnp.einsum(
        "bqhd,bkhd->bhqk", Q.astype(jnp.bfloat16), K_full,
        preferred_element_type=jnp.float32,
    ) * scale
    m = S.max(-1, keepdims=True)
    P = jnp.exp(S - m)
    P = P / P.sum(-1, keepdims=True)
    out = jnp.einsum(
        "bhqk,bkhd->bqhd", P.astype(jnp.bfloat16), V_full,
        preferred_element_type=jnp.float32,
    )
    return out.astype(jnp.float32)


# device time: 56101 ns/iter; 1.0614x vs baseline; 1.0614x over previous
import jax
import jax.numpy as jnp
from jax import lax
from jax.experimental import pallas as pl
from jax.experimental.pallas import tpu as pltpu

N_Y = 4


def _bidir_allgather_kv(Kf, Vf):
    b, s, f = Kf.shape
    hs = s // 2

    def body(k_ref, v_ref, ko_ref, vo_ref,
             lbuf_k, lbuf_v, rbuf_k, rbuf_v, fbuf_k, fbuf_v,
             send_sems, recv_sems):
        my_x = lax.axis_index("x")
        my_y = lax.axis_index("y")
        my_z = lax.axis_index("z")
        left_y = (my_y - 1) % N_Y
        right_y = (my_y + 1) % N_Y
        far_y = (my_y + 2) % N_Y
        left = (my_x, left_y, my_z)
        right = (my_x, right_y, my_z)

        barrier_sem = pltpu.get_barrier_semaphore()
        for nbr in (left, right):
            pl.semaphore_signal(
                barrier_sem, inc=1,
                device_id=nbr, device_id_type=pl.DeviceIdType.MESH,
            )
        pl.semaphore_wait(barrier_sem, 2)

        def rdma(i, src, dst, dev):
            return pltpu.make_async_remote_copy(
                src_ref=src, dst_ref=dst,
                send_sem=send_sems.at[i], recv_sem=recv_sems.at[i],
                device_id=dev, device_id_type=pl.DeviceIdType.MESH,
            )

        own_r_k = rdma(0, k_ref, lbuf_k, right)
        own_r_v = rdma(1, v_ref, lbuf_v, right)
        own_l_k = rdma(2, k_ref, rbuf_k, left)
        own_l_v = rdma(3, v_ref, rbuf_v, left)
        own_r_k.start()
        own_r_v.start()
        own_l_k.start()
        own_l_v.start()

        ko_ref[:, pl.ds(my_y * s, s), :] = k_ref[...]
        vo_ref[:, pl.ds(my_y * s, s), :] = v_ref[...]

        own_r_k.wait_recv()
        own_r_v.wait_recv()
        fwd_a_k = rdma(4, lbuf_k.at[:, 0:hs, :], fbuf_k.at[:, 0:hs, :], right)
        fwd_a_v = rdma(5, lbuf_v.at[:, 0:hs, :], fbuf_v.at[:, 0:hs, :], right)
        fwd_a_k.start()
        fwd_a_v.start()
        ko_ref[:, pl.ds(left_y * s, s), :] = lbuf_k[...]
        vo_ref[:, pl.ds(left_y * s, s), :] = lbuf_v[...]

        own_l_k.wait_recv()
        own_l_v.wait_recv()
        fwd_b_k = rdma(6, rbuf_k.at[:, hs:s, :], fbuf_k.at[:, hs:s, :], left)
        fwd_b_v = rdma(7, rbuf_v.at[:, hs:s, :], fbuf_v.at[:, hs:s, :], left)
        fwd_b_k.start()
        fwd_b_v.start()
        ko_ref[:, pl.ds(right_y * s, s), :] = rbuf_k[...]
        vo_ref[:, pl.ds(right_y * s, s), :] = rbuf_v[...]

        fwd_a_k.wait_recv()
        fwd_a_v.wait_recv()
        fwd_b_k.wait_recv()
        fwd_b_v.wait_recv()
        ko_ref[:, pl.ds(far_y * s, s), :] = fbuf_k[...]
        vo_ref[:, pl.ds(far_y * s, s), :] = fbuf_v[...]

        for r in (own_r_k, own_r_v, own_l_k, own_l_v,
                  fwd_a_k, fwd_a_v, fwd_b_k, fwd_b_v):
            r.wait_send()

    out_sd = jax.ShapeDtypeStruct((b, N_Y * s, f), Kf.dtype)
    chunk = lambda: pltpu.VMEM((b, s, f), Kf.dtype)
    return pl.pallas_call(
        body,
        out_shape=[out_sd, out_sd],
        in_specs=[
            pl.BlockSpec(memory_space=pltpu.VMEM),
            pl.BlockSpec(memory_space=pltpu.VMEM),
        ],
        out_specs=[
            pl.BlockSpec(memory_space=pltpu.VMEM),
            pl.BlockSpec(memory_space=pltpu.VMEM),
        ],
        scratch_shapes=[
            chunk(), chunk(),
            chunk(), chunk(),
            chunk(), chunk(),
            pltpu.SemaphoreType.DMA((8,)),
            pltpu.SemaphoreType.DMA((8,)),
        ],
        compiler_params=pltpu.CompilerParams(collective_id=0),
    )(Kf, Vf)


def kernel(Q, K, V):
    b, s, h, d = K.shape
    f = h * d

    Kf = K.astype(jnp.bfloat16).reshape(b, s, f)
    Vf = V.astype(jnp.bfloat16).reshape(b, s, f)
    K_full, V_full = _bidir_allgather_kv(Kf, Vf)
    K_full = K_full.reshape(b, N_Y * s, h, d)
    V_full = V_full.reshape(b, N_Y * s, h, d)

    scale = d ** -0.5
    S = jnp.einsum(
        "bqhd,bkhd->bhqk", Q.astype(jnp.bfloat16), K_full,
        preferred_element_type=jnp.float32,
    ) * scale
    m = S.max(-1, keepdims=True)
    P = jnp.exp(S - m)
    P = P / P.sum(-1, keepdims=True)
    out = jnp.einsum(
        "bhqk,bkhd->bqhd", P.astype(jnp.bfloat16), V_full,
        preferred_element_type=jnp.float32,
    )
    return out.astype(jnp.float32)


# device time: 34018 ns/iter; 1.7504x vs baseline; 1.6492x over previous
import jax
import jax.numpy as jnp
from jax import lax
from jax.experimental import pallas as pl
from jax.experimental.pallas import tpu as pltpu

N_Y = 4
H = 8
D = 64
HH = H // 2


def _fused_ag_attention(QT, KT, VT):
    _, b, s, fh = KT.shape
    hs = s // 2

    def body(q_ref, k_ref, v_ref, o_ref,
             lbuf_k, lbuf_v, rbuf_k, rbuf_v, fbuf_k, fbuf_v,
             osend, orecv, acc, lsum, send_sems, recv_sems):
        my_x = lax.axis_index("x")
        my_y = lax.axis_index("y")
        my_z = lax.axis_index("z")
        left = (my_x, (my_y - 1) % N_Y, my_z)
        right = (my_x, (my_y + 1) % N_Y, my_z)
        partner = (1 - my_x, my_y, my_z)

        barrier_sem = pltpu.get_barrier_semaphore()
        for nbr in (left, right, partner):
            pl.semaphore_signal(
                barrier_sem, inc=1,
                device_id=nbr, device_id_type=pl.DeviceIdType.MESH,
            )
        pl.semaphore_wait(barrier_sem, 3)

        def rdma(i, src, dst, dev):
            return pltpu.make_async_remote_copy(
                src_ref=src, dst_ref=dst,
                send_sem=send_sems.at[i], recv_sem=recv_sems.at[i],
                device_id=dev, device_id_type=pl.DeviceIdType.MESH,
            )

        dl_k = rdma(0, k_ref.at[my_x], lbuf_k, right)
        dl_v = rdma(1, v_ref.at[my_x], lbuf_v, right)
        dr_k = rdma(2, k_ref.at[my_x], rbuf_k, left)
        dr_v = rdma(3, v_ref.at[my_x], rbuf_v, left)
        dl_k.start()
        dl_v.start()
        dr_k.start()
        dr_v.start()

        acc[...] = jnp.zeros((b, s, fh), jnp.float32)
        lsum[...] = jnp.zeros((b, HH, s, 1), jnp.float32)

        def piece(kp_ref, vp_ref):
            for hh in range(HH):
                sl = slice(hh * D, (hh + 1) * D)
                Qh = q_ref[my_x, :, :, sl]
                Kh = kp_ref[:, :, sl]
                S = lax.dot_general(
                    Qh, Kh, (((2,), (2,)), ((0,), (0,))),
                    preferred_element_type=jnp.float32)
                P = jnp.exp(S)
                lsum[:, hh, :, :] = lsum[:, hh, :, :] + jnp.sum(
                    P, axis=2, keepdims=True)
                Vh = vp_ref[:, :, sl]
                O = lax.dot_general(
                    P.astype(jnp.bfloat16), Vh,
                    (((2,), (1,)), ((0,), (0,))),
                    preferred_element_type=jnp.float32)
                acc[:, :, sl] = acc[:, :, sl] + O

        piece(k_ref.at[my_x], v_ref.at[my_x])

        dl_k.wait_recv()
        dl_v.wait_recv()
        fa_k = rdma(4, lbuf_k.at[:, 0:hs, :], fbuf_k.at[:, 0:hs, :], right)
        fa_v = rdma(5, lbuf_v.at[:, 0:hs, :], fbuf_v.at[:, 0:hs, :], right)
        fa_k.start()
        fa_v.start()
        piece(lbuf_k, lbuf_v)

        dr_k.wait_recv()
        dr_v.wait_recv()
        fb_k = rdma(6, rbuf_k.at[:, hs:s, :], fbuf_k.at[:, hs:s, :], left)
        fb_v = rdma(7, rbuf_v.at[:, hs:s, :], fbuf_v.at[:, hs:s, :], left)
        fb_k.start()
        fb_v.start()
        piece(rbuf_k, rbuf_v)

        fa_k.wait_recv()
        fa_v.wait_recv()
        fb_k.wait_recv()
        fb_v.wait_recv()
        piece(fbuf_k, fbuf_v)

        for hh in range(HH):
            sl = slice(hh * D, (hh + 1) * D)
            On = acc[:, :, sl] / lsum[:, hh, :, :]
            o_ref[my_x, :, :, sl] = On
            osend[:, :, sl] = On.astype(jnp.bfloat16)

        sw = rdma(8, osend, orecv, partner)
        sw.start()
        sw.wait_recv()
        o_ref[1 - my_x] = orecv[...].astype(jnp.float32)

        for r in (dl_k, dl_v, dr_k, dr_v, fa_k, fa_v, fb_k, fb_v, sw):
            r.wait_send()

    cbuf = lambda: pltpu.VMEM((b, s, fh), KT.dtype)
    return pl.pallas_call(
        body,
        out_shape=jax.ShapeDtypeStruct((2, b, s, fh), jnp.float32),
        in_specs=[pl.BlockSpec(memory_space=pltpu.VMEM)] * 3,
        out_specs=pl.BlockSpec(memory_space=pltpu.VMEM),
        scratch_shapes=[
            cbuf(), cbuf(),
            cbuf(), cbuf(),
            cbuf(), cbuf(),
            cbuf(), cbuf(),
            pltpu.VMEM((b, s, fh), jnp.float32),
            pltpu.VMEM((b, HH, s, 1), jnp.float32),
            pltpu.SemaphoreType.DMA((9,)),
            pltpu.SemaphoreType.DMA((9,)),
        ],
        compiler_params=pltpu.CompilerParams(collective_id=0),
    )(QT, KT, VT)


def kernel(Q, K, V):
    b, s, h, d = K.shape
    f = h * d
    fh = f // 2
    scale = d ** -0.5

    def to_halves(A):
        return (A.astype(jnp.bfloat16)
                .reshape(b, s, 2, fh)
                .transpose(2, 0, 1, 3))

    QT = to_halves(Q * scale)
    KT = to_halves(K)
    VT = to_halves(V)
    O = _fused_ag_attention(QT, KT, VT)
    return O.transpose(1, 2, 0, 3).reshape(b, s, h, d)


# device time: 26044 ns/iter; 2.2863x vs baseline; 1.3062x over previous
import jax
import jax.numpy as jnp
from jax import lax
from jax.experimental import pallas as pl
from jax.experimental.pallas import tpu as pltpu

N_Y = 4
H = 8
D = 64
HQ = 2
FQ = HQ * D


def _fused_ag_attention(QT, KT, VT, scale):
    _, b, s, fq = KT.shape
    hs = s // 2

    def body(q_ref, k_ref, v_ref, o_ref,
             lbuf_k, lbuf_v, rbuf_k, rbuf_v, fbuf_k, fbuf_v,
             qbuf, osend, orx, orz, ord_,
             acc, lsum, send_sems, recv_sems):
        my_x = lax.axis_index("x")
        my_y = lax.axis_index("y")
        my_z = lax.axis_index("z")
        pz = jnp.bitwise_xor(my_z, 1)
        left = (my_x, (my_y - 1) % N_Y, my_z)
        right = (my_x, (my_y + 1) % N_Y, my_z)
        xpart = (1 - my_x, my_y, my_z)
        zpart = (my_x, my_y, pz)
        diag = (1 - my_x, my_y, pz)
        qi = 2 * my_x + (my_z % 2)

        barrier_sem = pltpu.get_barrier_semaphore()
        for nbr in (left, right, xpart, zpart, diag):
            pl.semaphore_signal(
                barrier_sem, inc=1,
                device_id=nbr, device_id_type=pl.DeviceIdType.MESH,
            )
        pl.semaphore_wait(barrier_sem, 5)

        def rdma(i, src, dst, dev):
            return pltpu.make_async_remote_copy(
                src_ref=src, dst_ref=dst,
                send_sem=send_sems.at[i], recv_sem=recv_sems.at[i],
                device_id=dev, device_id_type=pl.DeviceIdType.MESH,
            )

        dl_k = rdma(0, k_ref.at[qi], lbuf_k, right)
        dl_v = rdma(1, v_ref.at[qi], lbuf_v, right)
        dr_k = rdma(2, k_ref.at[qi], rbuf_k, left)
        dr_v = rdma(3, v_ref.at[qi], rbuf_v, left)
        dl_k.start()
        dl_v.start()
        dr_k.start()
        dr_v.start()

        qbuf[...] = q_ref[qi] * jnp.bfloat16(scale)
        acc[...] = jnp.zeros((b, s, fq), jnp.float32)
        lsum[...] = jnp.zeros((b, HQ, s, 1), jnp.float32)

        def piece(kp_ref, vp_ref):
            for hh in range(HQ):
                sl = slice(hh * D, (hh + 1) * D)
                Qh = qbuf[:, :, sl]
                Kh = kp_ref[:, :, sl]
                S = lax.dot_general(
                    Qh, Kh, (((2,), (2,)), ((0,), (0,))),
                    preferred_element_type=jnp.float32)
                P = jnp.exp(S)
                lsum[:, hh, :, :] = lsum[:, hh, :, :] + jnp.sum(
                    P, axis=2, keepdims=True)
                Vh = vp_ref[:, :, sl]
                O = lax.dot_general(
                    P.astype(jnp.bfloat16), Vh,
                    (((2,), (1,)), ((0,), (0,))),
                    preferred_element_type=jnp.float32)
                acc[:, :, sl] = acc[:, :, sl] + O

        piece(k_ref.at[qi], v_ref.at[qi])

        dl_k.wait_recv()
        dl_v.wait_recv()
        fa_k = rdma(4, lbuf_k.at[:, 0:hs, :], fbuf_k.at[:, 0:hs, :], right)
        fa_v = rdma(5, lbuf_v.at[:, 0:hs, :], fbuf_v.at[:, 0:hs, :], right)
        fa_k.start()
        fa_v.start()
        piece(lbuf_k, lbuf_v)

        dr_k.wait_recv()
        dr_v.wait_recv()
        fb_k = rdma(6, rbuf_k.at[:, hs:s, :], fbuf_k.at[:, hs:s, :], left)
        fb_v = rdma(7, rbuf_v.at[:, hs:s, :], fbuf_v.at[:, hs:s, :], left)
        fb_k.start()
        fb_v.start()
        piece(rbuf_k, rbuf_v)

        fa_k.wait_recv()
        fa_v.wait_recv()
        fb_k.wait_recv()
        fb_v.wait_recv()
        piece(fbuf_k, fbuf_v)

        On = jnp.concatenate(
            [acc[:, :, hh * D:(hh + 1) * D] / lsum[:, hh, :, :]
             for hh in range(HQ)], axis=2)
        o_ref[:, :, pl.ds(qi * fq, fq)] = On
        osend[...] = On.astype(jnp.bfloat16)

        swx = rdma(8, osend, orx, xpart)
        swz = rdma(9, osend, orz, zpart)
        swd = rdma(10, osend, ord_, diag)
        swx.start()
        swz.start()
        swd.start()

        qx = jnp.bitwise_xor(qi, 2)
        qz = jnp.bitwise_xor(qi, 1)
        qd = jnp.bitwise_xor(qi, 3)
        swx.wait_recv()
        o_ref[:, :, pl.ds(qx * fq, fq)] = orx[...].astype(jnp.float32)
        swz.wait_recv()
        o_ref[:, :, pl.ds(qz * fq, fq)] = orz[...].astype(jnp.float32)
        swd.wait_recv()
        o_ref[:, :, pl.ds(qd * fq, fq)] = ord_[...].astype(jnp.float32)

        for r in (dl_k, dl_v, dr_k, dr_v, fa_k, fa_v, fb_k, fb_v,
                  swx, swz, swd):
            r.wait_send()

    cbuf = lambda: pltpu.VMEM((b, s, fq), KT.dtype)
    return pl.pallas_call(
        body,
        out_shape=jax.ShapeDtypeStruct((b, s, 4 * fq), jnp.float32),
        in_specs=[pl.BlockSpec(memory_space=pltpu.VMEM)] * 3,
        out_specs=pl.BlockSpec(memory_space=pltpu.VMEM),
        scratch_shapes=[
            cbuf(), cbuf(),
            cbuf(), cbuf(),
            cbuf(), cbuf(),
            cbuf(),
            cbuf(), cbuf(), cbuf(), cbuf(),
            pltpu.VMEM((b, s, fq), jnp.float32),
            pltpu.VMEM((b, HQ, s, 1), jnp.float32),
            pltpu.SemaphoreType.DMA((11,)),
            pltpu.SemaphoreType.DMA((11,)),
        ],
        compiler_params=pltpu.CompilerParams(collective_id=0),
    )(QT, KT, VT)


def kernel(Q, K, V):
    b, s, h, d = K.shape
    f = h * d
    scale = d ** -0.5

    def to_quarters(A):
        return (A.astype(jnp.bfloat16)
                .reshape(b, s, 4, FQ)
                .transpose(2, 0, 1, 3))

    O = _fused_ag_attention(
        to_quarters(Q), to_quarters(K), to_quarters(V), scale)
    return O.reshape(b, s, h, d)


# device time: 25262 ns/iter; 2.3571x vs baseline; 1.0310x over previous
import jax
import jax.numpy as jnp
from jax import lax
from jax.experimental import pallas as pl
from jax.experimental.pallas import tpu as pltpu

N_Y = 4
H = 8
D = 64
HQ = 2
FQ = HQ * D


def _fused_ag_attention(QT, KT, VT, scale):
    _, b, s, fq = KT.shape
    hs = s // 2

    def body(q_ref, k_ref, v_ref, o_ref,
             lbuf_k, lbuf_v, rbuf_k, rbuf_v, fbuf_k, fbuf_v,
             qbuf, osend, orx, orz, ord_,
             acc, lsum, send_sems, recv_sems):
        my_x = lax.axis_index("x")
        my_y = lax.axis_index("y")
        my_z = lax.axis_index("z")
        pz = jnp.bitwise_xor(my_z, 1)
        left = (my_x, (my_y - 1) % N_Y, my_z)
        right = (my_x, (my_y + 1) % N_Y, my_z)
        xpart = (1 - my_x, my_y, my_z)
        zpart = (my_x, my_y, pz)
        diag = (1 - my_x, my_y, pz)
        qi = 2 * my_x + (my_z % 2)

        barrier_sem = pltpu.get_barrier_semaphore()
        for nbr in (left, right, xpart, zpart, diag):
            pl.semaphore_signal(
                barrier_sem, inc=1,
                device_id=nbr, device_id_type=pl.DeviceIdType.MESH,
            )
        pl.semaphore_wait(barrier_sem, 5)

        def rdma(i, src, dst, dev):
            return pltpu.make_async_remote_copy(
                src_ref=src, dst_ref=dst,
                send_sem=send_sems.at[i], recv_sem=recv_sems.at[i],
                device_id=dev, device_id_type=pl.DeviceIdType.MESH,
            )

        dl_k = rdma(0, k_ref.at[qi], lbuf_k, right)
        dl_v = rdma(1, v_ref.at[qi], lbuf_v, right)
        dr_k = rdma(2, k_ref.at[qi], rbuf_k, left)
        dr_v = rdma(3, v_ref.at[qi], rbuf_v, left)
        dl_k.start()
        dl_v.start()
        dr_k.start()
        dr_v.start()

        qbuf[...] = q_ref[qi] * jnp.bfloat16(scale)
        acc[...] = jnp.zeros((b, s, fq), jnp.float32)
        lsum[...] = jnp.zeros((b, HQ, s, 1), jnp.float32)

        def piece(kp_ref, vp_ref):
            for hh in range(HQ):
                sl = slice(hh * D, (hh + 1) * D)
                Qh = qbuf[:, :, sl]
                Kh = kp_ref[:, :, sl]
                S = lax.dot_general(
                    Qh, Kh, (((2,), (2,)), ((0,), (0,))),
                    preferred_element_type=jnp.float32)
                P = jnp.exp(S)
                lsum[:, hh, :, :] = lsum[:, hh, :, :] + jnp.sum(
                    P, axis=2, keepdims=True)
                Vh = vp_ref[:, :, sl]
                O = lax.dot_general(
                    P.astype(jnp.bfloat16), Vh,
                    (((2,), (1,)), ((0,), (0,))),
                    preferred_element_type=jnp.float32)
                acc[:, :, sl] = acc[:, :, sl] + O

        piece(k_ref.at[qi], v_ref.at[qi])

        dl_k.wait_recv()
        fa_k = rdma(4, lbuf_k.at[:, 0:hs, :], fbuf_k.at[:, 0:hs, :], right)
        fa_k.start()
        dl_v.wait_recv()
        fa_v = rdma(5, lbuf_v.at[:, 0:hs, :], fbuf_v.at[:, 0:hs, :], right)
        fa_v.start()
        piece(lbuf_k, lbuf_v)

        dr_k.wait_recv()
        fb_k = rdma(6, rbuf_k.at[:, hs:s, :], fbuf_k.at[:, hs:s, :], left)
        fb_k.start()
        dr_v.wait_recv()
        fb_v = rdma(7, rbuf_v.at[:, hs:s, :], fbuf_v.at[:, hs:s, :], left)
        fb_v.start()
        piece(rbuf_k, rbuf_v)

        fa_k.wait_recv()
        fa_v.wait_recv()
        fb_k.wait_recv()
        fb_v.wait_recv()
        piece(fbuf_k, fbuf_v)

        On = jnp.concatenate(
            [acc[:, :, hh * D:(hh + 1) * D] / lsum[:, hh, :, :]
             for hh in range(HQ)], axis=2)
        o_ref[:, :, pl.ds(qi * fq, fq)] = On
        osend[...] = On.astype(jnp.bfloat16)

        swx = rdma(8, osend, orx, xpart)
        swz = rdma(9, osend, orz, zpart)
        swd = rdma(10, osend, ord_, diag)
        swx.start()
        swz.start()
        swd.start()

        qx = jnp.bitwise_xor(qi, 2)
        qz = jnp.bitwise_xor(qi, 1)
        qd = jnp.bitwise_xor(qi, 3)
        swx.wait_recv()
        o_ref[:, :, pl.ds(qx * fq, fq)] = orx[...].astype(jnp.float32)
        swz.wait_recv()
        o_ref[:, :, pl.ds(qz * fq, fq)] = orz[...].astype(jnp.float32)
        swd.wait_recv()
        o_ref[:, :, pl.ds(qd * fq, fq)] = ord_[...].astype(jnp.float32)

        for r in (dl_k, dl_v, dr_k, dr_v, fa_k, fa_v, fb_k, fb_v,
                  swx, swz, swd):
            r.wait_send()

    cbuf = lambda: pltpu.VMEM((b, s, fq), KT.dtype)
    return pl.pallas_call(
        body,
        out_shape=jax.ShapeDtypeStruct((b, s, 4 * fq), jnp.float32),
        in_specs=[pl.BlockSpec(memory_space=pltpu.VMEM)] * 3,
        out_specs=pl.BlockSpec(memory_space=pltpu.VMEM),
        scratch_shapes=[
            cbuf(), cbuf(),
            cbuf(), cbuf(),
            cbuf(), cbuf(),
            cbuf(),
            cbuf(), cbuf(), cbuf(), cbuf(),
            pltpu.VMEM((b, s, fq), jnp.float32),
            pltpu.VMEM((b, HQ, s, 1), jnp.float32),
            pltpu.SemaphoreType.DMA((11,)),
            pltpu.SemaphoreType.DMA((11,)),
        ],
        compiler_params=pltpu.CompilerParams(collective_id=0),
    )(QT, KT, VT)


def kernel(Q, K, V):
    b, s, h, d = K.shape
    f = h * d
    scale = d ** -0.5

    def to_quarters(A):
        return (A.astype(jnp.bfloat16)
                .reshape(b, s, 4, FQ)
                .transpose(2, 0, 1, 3))

    O = _fused_ag_attention(
        to_quarters(Q), to_quarters(K), to_quarters(V), scale)
    return O.reshape(b, s, h, d)
